# Initial kernel scaffold; baseline (speedup 1.0000x reference)
import functools

import jax
import jax.numpy as jnp
from jax import lax
from jax.experimental import pallas as pl
from jax.experimental.pallas import tpu as pltpu
from jax.experimental.pallas import tpu_sc as plsc

_VOCAB = 128
_DIM = 2048
_NC = 2
_NS = 16
_NW = _NC * _NS
_K = 16
_MAGIC = 12582912.0


def _make_kernel(B):
    rows_per_w = B // _NW
    n_chunks = rows_per_w // _K
    mesh = plsc.VectorSubcoreMesh(core_axis_name="c", subcore_axis_name="s")

    @functools.partial(
        pl.kernel,
        out_type=jax.ShapeDtypeStruct((B, _DIM), jnp.float32),
        mesh=mesh,
        scratch_types=[
            pltpu.VMEM((4 * rows_per_w,), jnp.float32),
            pltpu.VMEM((_K,), jnp.int32),
            pltpu.VMEM((_K,), jnp.int32),
            pltpu.VMEM((_K, _DIM), jnp.float32),
            pltpu.VMEM((_K, _DIM), jnp.float32),
            pltpu.SemaphoreType.DMA,
        ],
    )
    def k(patch_hbm, rowtab_hbm, coltab_hbm, out_hbm,
          patch_v, idxr_v, idxc_v, bufr_v, bufc_v, sem):
        wid = lax.axis_index("s") * _NC + lax.axis_index("c")
        base_row = wid * rows_per_w
        pltpu.sync_copy(patch_hbm.at[pl.ds(base_row * 4, rows_per_w * 4)],
                        patch_v)
        stride4 = lax.iota(jnp.int32, 16) * 4

        def qidx(lo, hi):
            a = (lo * float(_VOCAB) + _MAGIC) - _MAGIC
            b = (hi * float(_VOCAB) + _MAGIC) - _MAGIC
            s = a.astype(jnp.int32) + b.astype(jnp.int32)
            i = lax.shift_right_logical(s, 1)
            return jnp.minimum(jnp.maximum(i, 0), _VOCAB - 1)

        @pl.loop(0, n_chunks)
        def chunk_loop(g):
            off = g * (4 * _K) + stride4
            rlo = plsc.load_gather(patch_v, [off])
            clo = plsc.load_gather(patch_v, [off + 1])
            rhi = plsc.load_gather(patch_v, [off + 2])
            chi = plsc.load_gather(patch_v, [off + 3])
            idxr_v[...] = qidx(rlo, rhi)
            idxc_v[...] = qidx(clo, chi)
            cr = pltpu.async_copy(rowtab_hbm.at[idxr_v], bufr_v, sem)
            cc = pltpu.async_copy(coltab_hbm.at[idxc_v], bufc_v, sem)
            cr.wait()
            cc.wait()

            @pl.loop(0, _DIM // 16)
            def add_loop(j):
                col = j * 16
                for i in range(_K):
                    bufr_v[i, pl.ds(col, 16)] = (
                        bufr_v[i, pl.ds(col, 16)] + bufc_v[i, pl.ds(col, 16)])

            pltpu.sync_copy(bufr_v, out_hbm.at[pl.ds(base_row + g * _K, _K)])

    return k


def kernel(patch_pos, row_embedding, column_embedding, eval=1):
    B = patch_pos.shape[0]
    patch_flat = patch_pos.reshape(-1)
    k = _make_kernel(B)
    return k(patch_flat, row_embedding, column_embedding)


# speedup vs baseline: 1.0614x; 1.0614x over previous
import functools

import jax
import jax.numpy as jnp
from jax import lax
from jax.experimental import pallas as pl
from jax.experimental.pallas import tpu as pltpu
from jax.experimental.pallas import tpu_sc as plsc

_VOCAB = 128
_DIM = 2048
_NC = 2
_NS = 16
_NW = _NC * _NS
_K = 16
_MAGIC = 12582912.0


def _make_kernel(B):
    rows_per_w = B // _NW
    n_chunks = rows_per_w // _K
    mesh = plsc.VectorSubcoreMesh(core_axis_name="c", subcore_axis_name="s")

    @functools.partial(
        pl.kernel,
        out_type=jax.ShapeDtypeStruct((B, _DIM), jnp.float32),
        mesh=mesh,
        scratch_types=[
            pltpu.VMEM((4, rows_per_w), jnp.float32),
            pltpu.VMEM((_K,), jnp.int32),
            pltpu.VMEM((_K,), jnp.int32),
            pltpu.VMEM((_K, _DIM), jnp.float32),
            pltpu.VMEM((_K, _DIM), jnp.float32),
            pltpu.SemaphoreType.DMA,
        ],
    )
    def k(patch_hbm, rowtab_hbm, coltab_hbm, out_hbm,
          patch_v, idxr_v, idxc_v, bufr_v, bufc_v, sem):
        wid = lax.axis_index("s") * _NC + lax.axis_index("c")
        base_row = wid * rows_per_w
        pltpu.sync_copy(patch_hbm.at[:, pl.ds(base_row, rows_per_w)],
                        patch_v)

        def qidx(lo, hi):
            a = (lo * float(_VOCAB) + _MAGIC) - _MAGIC
            b = (hi * float(_VOCAB) + _MAGIC) - _MAGIC
            s = a.astype(jnp.int32) + b.astype(jnp.int32)
            i = lax.shift_right_logical(s, 1)
            return jnp.minimum(jnp.maximum(i, 0), _VOCAB - 1)

        @pl.loop(0, n_chunks)
        def chunk_loop(g):
            off = g * _K
            rlo = patch_v[0, pl.ds(off, 16)]
            clo = patch_v[1, pl.ds(off, 16)]
            rhi = patch_v[2, pl.ds(off, 16)]
            chi = patch_v[3, pl.ds(off, 16)]
            idxr_v[...] = qidx(rlo, rhi)
            idxc_v[...] = qidx(clo, chi)
            cr = pltpu.async_copy(rowtab_hbm.at[idxr_v], bufr_v, sem)
            cc = pltpu.async_copy(coltab_hbm.at[idxc_v], bufc_v, sem)
            cr.wait()
            cc.wait()

            @pl.loop(0, _DIM // 16)
            def add_loop(j):
                col = j * 16
                for i in range(_K):
                    bufr_v[i, pl.ds(col, 16)] = (
                        bufr_v[i, pl.ds(col, 16)] + bufc_v[i, pl.ds(col, 16)])

            pltpu.sync_copy(bufr_v, out_hbm.at[pl.ds(base_row + g * _K, _K)])

    return k


def kernel(patch_pos, row_embedding, column_embedding, eval=1):
    B = patch_pos.shape[0]
    patch_t = patch_pos.reshape(B, 4).T
    k = _make_kernel(B)
    return k(patch_t, row_embedding, column_embedding)


# speedup vs baseline: 1.6333x; 1.5388x over previous
import functools

import jax
import jax.numpy as jnp
from jax import lax
from jax.experimental import pallas as pl
from jax.experimental.pallas import tpu as pltpu
from jax.experimental.pallas import tpu_sc as plsc

_VOCAB = 128
_DIM = 2048
_NC = 2
_NS = 16
_NW = _NC * _NS
_K = 8
_SETS = 3
_MAGIC = 12582912.0


def _make_kernel(B):
    rows_per_w = B // _NW
    n_chunks = rows_per_w // _K
    n_groups = rows_per_w // 16
    mesh = plsc.VectorSubcoreMesh(core_axis_name="c", subcore_axis_name="s")

    @functools.partial(
        pl.kernel,
        out_type=jax.ShapeDtypeStruct((B, _DIM), jnp.float32),
        mesh=mesh,
        scratch_types=[
            pltpu.VMEM((4, rows_per_w), jnp.float32),
            pltpu.VMEM((rows_per_w,), jnp.int32),
            pltpu.VMEM((rows_per_w,), jnp.int32),
            [pltpu.VMEM((_K, _DIM), jnp.float32) for _ in range(_SETS)],
            [pltpu.VMEM((_K, _DIM), jnp.float32) for _ in range(_SETS)],
            [pltpu.SemaphoreType.DMA for _ in range(_SETS)],
            [pltpu.SemaphoreType.DMA for _ in range(_SETS)],
        ],
    )
    def k(patch_hbm, rowtab_hbm, coltab_hbm, out_hbm,
          patch_v, idxr_v, idxc_v, bufr, bufc, gsem, osem):
        wid = lax.axis_index("s") * _NC + lax.axis_index("c")
        base_row = wid * rows_per_w
        pltpu.sync_copy(patch_hbm.at[:, pl.ds(base_row, rows_per_w)],
                        patch_v)

        def qidx(lo, hi):
            a = (lo * float(_VOCAB) + _MAGIC) - _MAGIC
            b = (hi * float(_VOCAB) + _MAGIC) - _MAGIC
            s = a.astype(jnp.int32) + b.astype(jnp.int32)
            i = lax.shift_right_logical(s, 1)
            return jnp.minimum(jnp.maximum(i, 0), _VOCAB - 1)

        @pl.loop(0, n_groups)
        def idx_loop(g):
            off = g * 16
            rlo = patch_v[0, pl.ds(off, 16)]
            clo = patch_v[1, pl.ds(off, 16)]
            rhi = patch_v[2, pl.ds(off, 16)]
            chi = patch_v[3, pl.ds(off, 16)]
            idxr_v[pl.ds(off, 16)] = qidx(rlo, rhi)
            idxc_v[pl.ds(off, 16)] = qidx(clo, chi)

        def start_gathers(c, s):
            pltpu.async_copy(
                rowtab_hbm.at[idxr_v.at[pl.ds(c * _K, _K)]], bufr[s], gsem[s])
            pltpu.async_copy(
                coltab_hbm.at[idxc_v.at[pl.ds(c * _K, _K)]], bufc[s], gsem[s])

        def wait_gathers(s):
            pltpu.make_async_copy(rowtab_hbm.at[idxr_v.at[pl.ds(0, _K)]],
                                  bufr[s], gsem[s]).wait()
            pltpu.make_async_copy(coltab_hbm.at[idxc_v.at[pl.ds(0, _K)]],
                                  bufc[s], gsem[s]).wait()

        def wait_out(s):
            pltpu.make_async_copy(bufr[s], out_hbm.at[pl.ds(base_row, _K)],
                                  osem[s]).wait()

        def accumulate(s):
            @pl.loop(0, _DIM // 16)
            def add_loop(j):
                col = j * 16
                for i in range(_K):
                    plsc.addupdate(bufr[s].at[i, pl.ds(col, 16)],
                                   bufc[s][i, pl.ds(col, 16)])

        def start_out(c, s):
            pltpu.async_copy(bufr[s], out_hbm.at[pl.ds(base_row + c * _K, _K)],
                             osem[s])

        start_gathers(0, 0)

        @pl.loop(0, n_chunks // _SETS)
        def pipe_loop(h):
            for kk in range(_SETS):
                s = kk
                s1 = (kk + 1) % _SETS
                c = h * _SETS + kk
                if kk == _SETS - 1:
                    wait_out(s1)
                else:
                    @pl.when(h > 0)
                    def _():
                        wait_out(s1)
                start_gathers(c + 1, s1)
                wait_gathers(s)
                accumulate(s)
                start_out(c, s)

        c_last = n_chunks - 1
        s_last = c_last % _SETS
        wait_gathers(s_last)
        accumulate(s_last)
        start_out(c_last, s_last)
        for s in range(_SETS):
            wait_out(s)

    return k


def kernel(patch_pos, row_embedding, column_embedding, eval=1):
    B = patch_pos.shape[0]
    patch_t = patch_pos.reshape(B, 4).T
    k = _make_kernel(B)
    return k(patch_t, row_embedding, column_embedding)
